# Initial kernel scaffold; baseline (speedup 1.0000x reference)
import jax
import jax.numpy as jnp
from jax import lax
from jax.experimental import pallas as pl
from jax.experimental.pallas import tpu as pltpu
from jax.experimental.pallas import tpu_sc as plsc

N, E, H = 10000, 160000, 256
C = 128
NCHUNK = 4
NC, NS = 2, 16
EPT = E // NS
BB = 80
NBLK = EPT // BB
EPT_D = E // (NC * NS)
BD = 40
NBLK_D = EPT_D // BD
RPT = N // NS
ZR = 125
DW = 16
PREC = lax.Precision.HIGHEST


def _fill2d(ref, rows, cols, value, dtype):
  vec = jnp.full((16,), value, dtype)

  def body(i, _):
    for j in range(cols // 16):
      ref[i, pl.ds(j * 16, 16)] = vec
    return 0

  lax.fori_loop(0, rows, body, 0)


def _sc_body(t_hbm, src_hbm, dst_hbm, acc_out, deg_out,
             idx_src, idx_dst, rows_v, idx_dst_d, ones_v, zbuf, dzbuf,
             acc_s, dacc_s, sem):
  c = lax.axis_index("c")
  s = lax.axis_index("s")

  _fill2d(zbuf, ZR, C, 0.0, jnp.float32)
  _fill2d(dzbuf, ZR, DW, 0.0, jnp.float32)
  _fill2d(ones_v, BD, DW, 1.0, jnp.float32)

  for p in range(2):
    chunk = 2 * c + p

    for k in range(RPT // ZR):
      pltpu.sync_copy(zbuf, acc_s.at[pl.ds(s * RPT + k * ZR, ZR)])
    plsc.subcore_barrier()

    def edge_body(i, _):
      base = s * EPT + i * BB
      pltpu.sync_copy(src_hbm.at[pl.ds(base, BB)], idx_src)
      pltpu.sync_copy(dst_hbm.at[pl.ds(base, BB)], idx_dst)
      pltpu.async_copy(t_hbm.at[chunk].at[idx_src], rows_v, sem).wait()
      pltpu.sync_copy(rows_v, acc_s.at[idx_dst], add=True)
      return 0

    lax.fori_loop(0, NBLK, edge_body, 0)
    plsc.subcore_barrier()

    for k in range(RPT // ZR):
      r = s * RPT + k * ZR
      pltpu.sync_copy(acc_s.at[pl.ds(r, ZR)], acc_out.at[chunk].at[pl.ds(r, ZR)])
    plsc.subcore_barrier()

  for k in range(RPT // ZR):
    pltpu.sync_copy(dzbuf, dacc_s.at[pl.ds(s * RPT + k * ZR, ZR)])
  plsc.subcore_barrier()

  def deg_body(i, _):
    base = c * (E // NC) + s * EPT_D + i * BD
    pltpu.sync_copy(dst_hbm.at[pl.ds(base, BD)], idx_dst_d)
    pltpu.sync_copy(ones_v, dacc_s.at[idx_dst_d], add=True)
    return 0

  lax.fori_loop(0, NBLK_D, deg_body, 0)
  plsc.subcore_barrier()

  for k in range(RPT // ZR):
    r = s * RPT + k * ZR
    pltpu.sync_copy(dacc_s.at[pl.ds(r, ZR)], deg_out.at[c].at[pl.ds(r, ZR)])


_sc_segsum = pl.kernel(
    _sc_body,
    out_type=(
        jax.ShapeDtypeStruct((NCHUNK, N, C), jnp.float32),
        jax.ShapeDtypeStruct((NC, N, DW), jnp.float32),
    ),
    mesh=plsc.VectorSubcoreMesh(core_axis_name="c", subcore_axis_name="s"),
    scratch_types=[
        pltpu.VMEM((BB,), jnp.int32),
        pltpu.VMEM((BB,), jnp.int32),
        pltpu.VMEM((BB, C), jnp.float32),
        pltpu.VMEM((BD,), jnp.int32),
        pltpu.VMEM((BD, DW), jnp.float32),
        pltpu.VMEM((ZR, C), jnp.float32),
        pltpu.VMEM((ZR, DW), jnp.float32),
        pltpu.VMEM_SHARED((N, C), jnp.float32),
        pltpu.VMEM_SHARED((N, DW), jnp.float32),
        pltpu.SemaphoreType.DMA,
    ],
)


BN = 1000


def _table_body(h_ref, c_ref, ufw_ref, ufb_ref, t_ref):
  h = h_ref[...]
  f = jax.nn.sigmoid(
      lax.dot_general(h, ufw_ref[...], (((1,), (1,)), ((), ())),
                      precision=PREC) + ufb_ref[...])
  p = f * c_ref[...]
  t_ref[0] = h[:, :C]
  t_ref[1] = h[:, C:]
  t_ref[2] = p[:, :C]
  t_ref[3] = p[:, C:]


def _gates_body(x_ref, c_ref, acc_ref, deg_ref, wiou_ref, uiou_ref, biou_ref,
                h_out, c_out):
  acc = acc_ref[...]
  h_tild = jnp.concatenate([acc[0], acc[1]], axis=1)
  c_agg = jnp.concatenate([acc[2], acc[3]], axis=1)
  deg = deg_ref[0, :, :1] + deg_ref[1, :, :1]
  has_child = deg > 0.0
  uh = lax.dot_general(h_tild, uiou_ref[...], (((1,), (1,)), ((), ())),
                       precision=PREC)
  wx = lax.dot_general(x_ref[...], wiou_ref[...], (((1,), (1,)), ((), ())),
                       precision=PREC)
  iou = jnp.where(has_child, uh, wx) + biou_ref[...]
  i_g = jax.nn.sigmoid(iou[:, :H])
  o_g = jax.nn.sigmoid(iou[:, H:2 * H])
  u_g = jnp.tanh(iou[:, 2 * H:])
  c_data = jnp.where(has_child, c_agg, c_ref[...])
  c_new = i_g * u_g + c_data
  h_out[...] = o_g * jnp.tanh(c_new)
  c_out[...] = c_new


def kernel(x, h, c, edge_index, W_iou, U_iou, b_iou, U_f_w, U_f_b):
  src = edge_index[0]
  dst = edge_index[1]
  ufb = U_f_b.reshape(1, H)

  table = pl.pallas_call(
      _table_body,
      grid=(N // BN,),
      in_specs=[
          pl.BlockSpec((BN, H), lambda i: (i, 0)),
          pl.BlockSpec((BN, H), lambda i: (i, 0)),
          pl.BlockSpec((H, H), lambda i: (0, 0)),
          pl.BlockSpec((1, H), lambda i: (0, 0)),
      ],
      out_specs=pl.BlockSpec((NCHUNK, BN, C), lambda i: (0, i, 0)),
      out_shape=jax.ShapeDtypeStruct((NCHUNK, N, C), jnp.float32),
  )(h, c, U_f_w, ufb)

  acc, degp = _sc_segsum(table, src, dst)

  h_new, c_new = pl.pallas_call(
      _gates_body,
      grid=(N // BN,),
      in_specs=[
          pl.BlockSpec((BN, H), lambda i: (i, 0)),
          pl.BlockSpec((BN, H), lambda i: (i, 0)),
          pl.BlockSpec((NCHUNK, BN, C), lambda i: (0, i, 0)),
          pl.BlockSpec((NC, BN, DW), lambda i: (0, i, 0)),
          pl.BlockSpec((3 * H, H), lambda i: (0, 0)),
          pl.BlockSpec((3 * H, H), lambda i: (0, 0)),
          pl.BlockSpec((1, 3 * H), lambda i: (0, 0)),
      ],
      out_specs=[
          pl.BlockSpec((BN, H), lambda i: (i, 0)),
          pl.BlockSpec((BN, H), lambda i: (i, 0)),
      ],
      out_shape=[
          jax.ShapeDtypeStruct((N, H), jnp.float32),
          jax.ShapeDtypeStruct((N, H), jnp.float32),
      ],
  )(x, c, acc, degp, W_iou, U_iou, b_iou)

  return (h_new, c_new)


# speedup vs baseline: 2.4995x; 2.4995x over previous
import jax
import jax.numpy as jnp
from jax import lax
from jax.experimental import pallas as pl
from jax.experimental.pallas import tpu as pltpu
from jax.experimental.pallas import tpu_sc as plsc

N, E, H = 10000, 160000, 256
C = 128
NCHUNK = 4
NC, NS = 2, 16
EPT = E // NS
BB = 80
NBLK = EPT // BB
NP = 10240
RPT = NP // NS
ZR = 80
DW = 16
PREC = lax.Precision.HIGHEST


def _fill2d(ref, rows, cols, value, dtype):
  vec = jnp.full((16,), value, dtype)

  def body(i, _):
    for j in range(cols // 16):
      ref[i, pl.ds(j * 16, 16)] = vec
    return 0

  lax.fori_loop(0, rows, body, 0)


def _sc_body(t_hbm, src_hbm, dst_hbm, acc_out, deg_out,
             idx_src, idx_dst, rows_v, ones_v, zbuf,
             acc_s, sem):
  c = lax.axis_index("c")
  s = lax.axis_index("s")

  _fill2d(zbuf, ZR, C, 0.0, jnp.float32)
  _fill2d(ones_v, BB, C, 1.0, jnp.float32)

  def do_chunk(chunk):
    for k in range(RPT // ZR):
      pltpu.sync_copy(zbuf, acc_s.at[pl.ds(s * RPT + k * ZR, ZR)])
    plsc.subcore_barrier()

    def edge_body(i, _):
      base = s * EPT + i * BB
      pltpu.sync_copy(src_hbm.at[pl.ds(base, BB)], idx_src)
      pltpu.sync_copy(dst_hbm.at[pl.ds(base, BB)], idx_dst)
      pltpu.async_copy(t_hbm.at[chunk].at[idx_src], rows_v, sem).wait()
      pltpu.sync_copy(rows_v, acc_s.at[idx_dst], add=True)
      return 0

    lax.fori_loop(0, NBLK, edge_body, 0)
    plsc.subcore_barrier()

    for k in range(RPT // ZR):
      r = s * RPT + k * ZR
      pltpu.sync_copy(acc_s.at[pl.ds(r, ZR)], rows_v)
      pltpu.sync_copy(rows_v, acc_out.at[chunk].at[pl.ds(r, ZR)])
    plsc.subcore_barrier()

  @pl.when(c == 0)
  def _():
    do_chunk(0)
    do_chunk(1)

  @pl.when(c == 1)
  def _():
    do_chunk(2)
    do_chunk(3)

  for k in range(RPT // ZR):
    pltpu.sync_copy(zbuf, acc_s.at[pl.ds(s * RPT + k * ZR, ZR)])
  plsc.subcore_barrier()

  def deg_body(i, _):
    base = s * EPT + i * BB
    pltpu.sync_copy(dst_hbm.at[pl.ds(base, BB)], idx_dst)
    pltpu.sync_copy(ones_v, acc_s.at[idx_dst], add=True)
    return 0

  lax.fori_loop(0, NBLK, deg_body, 0)
  plsc.subcore_barrier()

  def deg_writeback(core):
    for k in range(RPT // ZR):
      r = s * RPT + k * ZR
      pltpu.sync_copy(acc_s.at[pl.ds(r, ZR)], rows_v)
      pltpu.sync_copy(rows_v, deg_out.at[core].at[pl.ds(r, ZR)])

  @pl.when(c == 0)
  def _():
    deg_writeback(0)

  @pl.when(c == 1)
  def _():
    deg_writeback(1)


def _make_sc_segsum():
  return pl.kernel(
    _sc_body,
    out_type=(
        jax.ShapeDtypeStruct((NCHUNK, NP, C), jnp.float32),
        jax.ShapeDtypeStruct((NC, NP, C), jnp.float32),
    ),
    mesh=plsc.VectorSubcoreMesh(
        core_axis_name="c", subcore_axis_name="s",
        num_cores=NC, num_subcores=NS),
    scratch_types=[
        pltpu.VMEM((BB,), jnp.int32),
        pltpu.VMEM((BB,), jnp.int32),
        pltpu.VMEM((BB, C), jnp.float32),
        pltpu.VMEM((BB, C), jnp.float32),
        pltpu.VMEM((ZR, C), jnp.float32),
        pltpu.VMEM_SHARED((NP, C), jnp.float32),
        pltpu.SemaphoreType.DMA,
    ],
  )


BN = 1000


def _table_body(h_ref, c_ref, ufw_ref, ufb_ref, t_ref):
  h = h_ref[...]
  f = jax.nn.sigmoid(
      lax.dot_general(h, ufw_ref[...], (((1,), (1,)), ((), ())),
                      precision=PREC) + ufb_ref[...])
  p = f * c_ref[...]
  t_ref[0] = h[:, :C]
  t_ref[1] = h[:, C:]
  t_ref[2] = p[:, :C]
  t_ref[3] = p[:, C:]


def _gates_body(x_ref, c_ref, acc_ref, deg_ref, wiou_ref, uiou_ref, biou_ref,
                h_out, c_out):
  acc = acc_ref[...]
  h_tild = jnp.concatenate([acc[0], acc[1]], axis=1)
  c_agg = jnp.concatenate([acc[2], acc[3]], axis=1)
  deg = deg_ref[0, :, :1] + deg_ref[1, :, :1]
  has_child = deg > 0.0
  uh = lax.dot_general(h_tild, uiou_ref[...], (((1,), (1,)), ((), ())),
                       precision=PREC)
  wx = lax.dot_general(x_ref[...], wiou_ref[...], (((1,), (1,)), ((), ())),
                       precision=PREC)
  iou = jnp.where(has_child, uh, wx) + biou_ref[...]
  i_g = jax.nn.sigmoid(iou[:, :H])
  o_g = jax.nn.sigmoid(iou[:, H:2 * H])
  u_g = jnp.tanh(iou[:, 2 * H:])
  c_data = jnp.where(has_child, c_agg, c_ref[...])
  c_new = i_g * u_g + c_data
  h_out[...] = o_g * jnp.tanh(c_new)
  c_out[...] = c_new


def kernel(x, h, c, edge_index, W_iou, U_iou, b_iou, U_f_w, U_f_b):
  src = edge_index[0]
  dst = edge_index[1]
  ufb = U_f_b.reshape(1, H)

  table = pl.pallas_call(
      _table_body,
      grid=(N // BN,),
      in_specs=[
          pl.BlockSpec((BN, H), lambda i: (i, 0)),
          pl.BlockSpec((BN, H), lambda i: (i, 0)),
          pl.BlockSpec((H, H), lambda i: (0, 0)),
          pl.BlockSpec((1, H), lambda i: (0, 0)),
      ],
      out_specs=pl.BlockSpec((NCHUNK, BN, C), lambda i: (0, i, 0)),
      out_shape=jax.ShapeDtypeStruct((NCHUNK, N, C), jnp.float32),
  )(h, c, U_f_w, ufb)

  acc, degp = _make_sc_segsum()(table, src, dst)

  h_new, c_new = pl.pallas_call(
      _gates_body,
      grid=(N // BN,),
      in_specs=[
          pl.BlockSpec((BN, H), lambda i: (i, 0)),
          pl.BlockSpec((BN, H), lambda i: (i, 0)),
          pl.BlockSpec((NCHUNK, BN, C), lambda i: (0, i, 0)),
          pl.BlockSpec((NC, BN, C), lambda i: (0, i, 0)),
          pl.BlockSpec((3 * H, H), lambda i: (0, 0)),
          pl.BlockSpec((3 * H, H), lambda i: (0, 0)),
          pl.BlockSpec((1, 3 * H), lambda i: (0, 0)),
      ],
      out_specs=[
          pl.BlockSpec((BN, H), lambda i: (i, 0)),
          pl.BlockSpec((BN, H), lambda i: (i, 0)),
      ],
      out_shape=[
          jax.ShapeDtypeStruct((N, H), jnp.float32),
          jax.ShapeDtypeStruct((N, H), jnp.float32),
      ],
  )(x, c, acc, degp, W_iou, U_iou, b_iou)

  return (h_new, c_new)


# speedup vs baseline: 5.8338x; 2.3340x over previous
import jax
import jax.numpy as jnp
from jax import lax
from jax.experimental import pallas as pl
from jax.experimental.pallas import tpu as pltpu
from jax.experimental.pallas import tpu_sc as plsc

N, E, H = 10000, 160000, 256
C = 128
NCHUNK = 4
NC, NS = 2, 16
EPT = E // NS
BB = 80
NBLK = EPT // BB
NSEG = 5
SB = NBLK // NSEG
SEGE = EPT // NSEG
NP = 10240
RPT = NP // NS
ZR = 80
DW = 16
PREC = lax.Precision.HIGHEST


def _fill2d(ref, rows, cols, value, dtype):
  vec = jnp.full((16,), value, dtype)

  def body(i, _):
    for j in range(cols // 16):
      ref[i, pl.ds(j * 16, 16)] = vec
    return 0

  lax.fori_loop(0, rows, body, 0)


def _sc_body(t_hbm, src_hbm, dst_hbm, acc_out, deg_out,
             src_a, dst_a, src_b, dst_b, sidx, rows0, rows1,
             acc_s, sem0, sem1, semi_a, semi_b):
  c = lax.axis_index("c")
  s = lax.axis_index("s")
  ibufs = ((src_a, dst_a, semi_a), (src_b, dst_b, semi_b))

  def idx_start(seg, buf):
    base = s * EPT + seg * SEGE
    srcb, dstb, semi = buf
    pltpu.async_copy(src_hbm.at[pl.ds(base, SEGE)], srcb, semi)
    pltpu.async_copy(dst_hbm.at[pl.ds(base, SEGE)], dstb, semi)

  def idx_wait(buf):
    srcb, dstb, semi = buf
    pltpu.make_async_copy(src_hbm.at[pl.ds(0, SEGE)], srcb, semi).wait()
    pltpu.make_async_copy(dst_hbm.at[pl.ds(0, SEGE)], dstb, semi).wait()

  def stage_sidx(dstb, j):
    for k in range(BB // 16):
      sidx[pl.ds(16 * k, 16)] = dstb[pl.ds(j * BB + 16 * k, 16)]

  def zero_acc():
    _fill2d(rows0, ZR, C, 0.0, jnp.float32)
    for k in range(RPT // ZR):
      pltpu.sync_copy(rows0, acc_s.at[pl.ds(s * RPT + k * ZR, ZR)])
    plsc.subcore_barrier()

  def writeback(out_ref):
    for k in range(RPT // ZR):
      r = s * RPT + k * ZR
      pltpu.sync_copy(acc_s.at[pl.ds(r, ZR)], rows0)
      pltpu.sync_copy(rows0, out_ref.at[pl.ds(r, ZR)])
    plsc.subcore_barrier()

  def do_chunk(chunk):
    table = t_hbm.at[chunk]

    def gather_start(srcb, j, rows, sem):
      pltpu.async_copy(table.at[srcb.at[pl.ds(j * BB, BB)]], rows, sem)

    def gather_wait(rows, sem):
      pltpu.make_async_copy(table.at[src_a.at[pl.ds(0, BB)]], rows, sem).wait()

    zero_acc()
    idx_start(0, ibufs[0])

    for seg in range(NSEG):
      cur = ibufs[seg % 2]
      nxt = ibufs[(seg + 1) % 2]
      idx_wait(cur)
      if seg + 1 < NSEG:
        idx_start(seg + 1, nxt)
      srcb, dstb = cur[0], cur[1]

      gather_start(srcb, 0, rows0, sem0)
      gather_start(srcb, 1, rows1, sem1)

      def pair_body(g, _):
        for b, (rows, sem) in enumerate(((rows0, sem0), (rows1, sem1))):
          j = 2 * g + b
          gather_wait(rows, sem)
          gather_start(srcb, jnp.minimum(j + 2, SB - 1), rows, sem)
          stage_sidx(dstb, j)
          pltpu.sync_copy(rows, acc_s.at[sidx], add=True)
        return 0

      lax.fori_loop(0, (SB - 1) // 2, pair_body, 0)
      gather_wait(rows0, sem0)
      stage_sidx(dstb, SB - 1)
      pltpu.sync_copy(rows0, acc_s.at[sidx], add=True)
      gather_wait(rows1, sem1)

    plsc.subcore_barrier()
    writeback(acc_out.at[chunk])

  @pl.when(c == 0)
  def _():
    do_chunk(0)
    do_chunk(1)

  @pl.when(c == 1)
  def _():
    do_chunk(2)
    do_chunk(3)

  zero_acc()
  _fill2d(rows1, BB, C, 1.0, jnp.float32)

  def do_deg_seg(seg):
    buf = ibufs[0]
    idx_start(seg, buf)
    idx_wait(buf)
    dstb = buf[1]

    def deg_body(j, _):
      stage_sidx(dstb, j)
      pltpu.sync_copy(rows1, acc_s.at[sidx], add=True)
      return 0

    lax.fori_loop(0, SB, deg_body, 0)

  @pl.when(c == 0)
  def _():
    for seg in (0, 1, 2):
      do_deg_seg(seg)

  @pl.when(c == 1)
  def _():
    for seg in (2, 3, 4):
      do_deg_seg(seg)

  plsc.subcore_barrier()

  @pl.when(c == 0)
  def _():
    writeback(deg_out.at[0])

  @pl.when(c == 1)
  def _():
    writeback(deg_out.at[1])


def _make_sc_segsum():
  return pl.kernel(
    _sc_body,
    out_type=(
        jax.ShapeDtypeStruct((NCHUNK, NP, C), jnp.float32),
        jax.ShapeDtypeStruct((NC, NP, C), jnp.float32),
    ),
    mesh=plsc.VectorSubcoreMesh(
        core_axis_name="c", subcore_axis_name="s",
        num_cores=NC, num_subcores=NS),
    scratch_types=[
        pltpu.VMEM((SEGE,), jnp.int32),
        pltpu.VMEM((SEGE,), jnp.int32),
        pltpu.VMEM((SEGE,), jnp.int32),
        pltpu.VMEM((SEGE,), jnp.int32),
        pltpu.VMEM((BB,), jnp.int32),
        pltpu.VMEM((BB, C), jnp.float32),
        pltpu.VMEM((BB, C), jnp.float32),
        pltpu.VMEM_SHARED((NP, C), jnp.float32),
        pltpu.SemaphoreType.DMA,
        pltpu.SemaphoreType.DMA,
        pltpu.SemaphoreType.DMA,
        pltpu.SemaphoreType.DMA,
    ],
  )


BN = 1000


def _table_body(h_ref, c_ref, ufw_ref, ufb_ref, t_ref):
  h = h_ref[...]
  f = jax.nn.sigmoid(
      lax.dot_general(h, ufw_ref[...], (((1,), (1,)), ((), ())),
                      precision=PREC) + ufb_ref[...])
  p = f * c_ref[...]
  t_ref[0] = h[:, :C]
  t_ref[1] = h[:, C:]
  t_ref[2] = p[:, :C]
  t_ref[3] = p[:, C:]


def _gates_body(x_ref, c_ref, acc_ref, deg_ref, wiou_ref, uiou_ref, biou_ref,
                h_out, c_out):
  acc = acc_ref[...]
  h_tild = jnp.concatenate([acc[0], acc[1]], axis=1)
  c_agg = jnp.concatenate([acc[2], acc[3]], axis=1)
  deg = deg_ref[0, :, :1] + deg_ref[1, :, :1]
  has_child = deg > 0.0
  uh = lax.dot_general(h_tild, uiou_ref[...], (((1,), (1,)), ((), ())),
                       precision=PREC)
  wx = lax.dot_general(x_ref[...], wiou_ref[...], (((1,), (1,)), ((), ())),
                       precision=PREC)
  iou = jnp.where(has_child, uh, wx) + biou_ref[...]
  i_g = jax.nn.sigmoid(iou[:, :H])
  o_g = jax.nn.sigmoid(iou[:, H:2 * H])
  u_g = jnp.tanh(iou[:, 2 * H:])
  c_data = jnp.where(has_child, c_agg, c_ref[...])
  c_new = i_g * u_g + c_data
  h_out[...] = o_g * jnp.tanh(c_new)
  c_out[...] = c_new


def kernel(x, h, c, edge_index, W_iou, U_iou, b_iou, U_f_w, U_f_b):
  src = edge_index[0]
  dst = edge_index[1]
  ufb = U_f_b.reshape(1, H)

  table = pl.pallas_call(
      _table_body,
      grid=(N // BN,),
      in_specs=[
          pl.BlockSpec((BN, H), lambda i: (i, 0)),
          pl.BlockSpec((BN, H), lambda i: (i, 0)),
          pl.BlockSpec((H, H), lambda i: (0, 0)),
          pl.BlockSpec((1, H), lambda i: (0, 0)),
      ],
      out_specs=pl.BlockSpec((NCHUNK, BN, C), lambda i: (0, i, 0)),
      out_shape=jax.ShapeDtypeStruct((NCHUNK, N, C), jnp.float32),
  )(h, c, U_f_w, ufb)

  acc, degp = _make_sc_segsum()(table, src, dst)

  h_new, c_new = pl.pallas_call(
      _gates_body,
      grid=(N // BN,),
      in_specs=[
          pl.BlockSpec((BN, H), lambda i: (i, 0)),
          pl.BlockSpec((BN, H), lambda i: (i, 0)),
          pl.BlockSpec((NCHUNK, BN, C), lambda i: (0, i, 0)),
          pl.BlockSpec((NC, BN, C), lambda i: (0, i, 0)),
          pl.BlockSpec((3 * H, H), lambda i: (0, 0)),
          pl.BlockSpec((3 * H, H), lambda i: (0, 0)),
          pl.BlockSpec((1, 3 * H), lambda i: (0, 0)),
      ],
      out_specs=[
          pl.BlockSpec((BN, H), lambda i: (i, 0)),
          pl.BlockSpec((BN, H), lambda i: (i, 0)),
      ],
      out_shape=[
          jax.ShapeDtypeStruct((N, H), jnp.float32),
          jax.ShapeDtypeStruct((N, H), jnp.float32),
      ],
  )(x, c, acc, degp, W_iou, U_iou, b_iou)

  return (h_new, c_new)


# speedup vs baseline: 6.4578x; 1.1070x over previous
import jax
import jax.numpy as jnp
from jax import lax
from jax.experimental import pallas as pl
from jax.experimental.pallas import tpu as pltpu
from jax.experimental.pallas import tpu_sc as plsc

N, E, H = 10000, 160000, 256
C = 128
NCHUNK = 4
NC, NS = 2, 16
EPT = E // NS
BB = 80
NBLK = EPT // BB
NSEG = 5
SB = NBLK // NSEG
SEGE = EPT // NSEG
NP = 10240
RPT = NP // NS
ZR = 80
DW = 16
PREC = lax.Precision.DEFAULT


def _fill2d(ref, rows, cols, value, dtype):
  vec = jnp.full((16,), value, dtype)

  def body(i, _):
    for j in range(cols // 16):
      ref[i, pl.ds(j * 16, 16)] = vec
    return 0

  lax.fori_loop(0, rows, body, 0)


def _sc_body(t_hbm, src_hbm, dst_hbm, acc_out, deg_out,
             src_a, dst_a, src_b, dst_b, sidx, rows0, rows1,
             acc_s, sem0, sem1, semi_a, semi_b):
  c = lax.axis_index("c")
  s = lax.axis_index("s")
  ibufs = ((src_a, dst_a, semi_a), (src_b, dst_b, semi_b))

  def idx_start(seg, buf):
    base = s * EPT + seg * SEGE
    srcb, dstb, semi = buf
    pltpu.async_copy(src_hbm.at[pl.ds(base, SEGE)], srcb, semi)
    pltpu.async_copy(dst_hbm.at[pl.ds(base, SEGE)], dstb, semi)

  def idx_wait(buf):
    srcb, dstb, semi = buf
    pltpu.make_async_copy(src_hbm.at[pl.ds(0, SEGE)], srcb, semi).wait()
    pltpu.make_async_copy(dst_hbm.at[pl.ds(0, SEGE)], dstb, semi).wait()

  def stage_sidx(dstb, j):
    for k in range(BB // 16):
      sidx[pl.ds(16 * k, 16)] = dstb[pl.ds(j * BB + 16 * k, 16)]

  def zero_acc():
    _fill2d(rows0, ZR, C, 0.0, jnp.float32)
    for k in range(RPT // ZR):
      pltpu.sync_copy(rows0.at[pl.ds(0, ZR)],
                      acc_s.at[pl.ds(s * RPT + k * ZR, ZR)])
    plsc.subcore_barrier()

  def writeback(out_ref):
    for k in range(RPT // ZR):
      r = s * RPT + k * ZR
      pltpu.sync_copy(acc_s.at[pl.ds(r, ZR)], rows0.at[pl.ds(0, ZR)])
      pltpu.sync_copy(rows0.at[pl.ds(0, ZR)], out_ref.at[pl.ds(r, ZR)])
    plsc.subcore_barrier()

  def do_chunk(chunk):
    table = t_hbm.at[chunk]

    def gather_start(srcb, j, rows, sem):
      pltpu.async_copy(table.at[srcb.at[pl.ds(j * BB, BB)]], rows, sem)

    def gather_wait(rows, sem):
      pltpu.make_async_copy(table.at[src_a.at[pl.ds(0, BB)]], rows, sem).wait()

    zero_acc()
    idx_start(0, ibufs[0])

    for seg in range(NSEG):
      cur = ibufs[seg % 2]
      nxt = ibufs[(seg + 1) % 2]
      idx_wait(cur)
      if seg + 1 < NSEG:
        idx_start(seg + 1, nxt)
      srcb, dstb = cur[0], cur[1]

      gather_start(srcb, 0, rows0, sem0)
      gather_start(srcb, 1, rows1, sem1)

      def pair_body(g, _):
        for b, (rows, sem) in enumerate(((rows0, sem0), (rows1, sem1))):
          j = 2 * g + b
          gather_wait(rows, sem)
          gather_start(srcb, jnp.minimum(j + 2, SB - 1), rows, sem)
          stage_sidx(dstb, j)
          pltpu.sync_copy(rows, acc_s.at[sidx], add=True)
        return 0

      lax.fori_loop(0, (SB - 1) // 2, pair_body, 0)
      gather_wait(rows0, sem0)
      stage_sidx(dstb, SB - 1)
      pltpu.sync_copy(rows0, acc_s.at[sidx], add=True)
      gather_wait(rows1, sem1)

    plsc.subcore_barrier()
    writeback(acc_out.at[chunk])

  @pl.when(c == 0)
  def _():
    do_chunk(0)
    do_chunk(1)

  @pl.when(c == 1)
  def _():
    do_chunk(2)
    do_chunk(3)

  zero_acc()
  _fill2d(rows1, BB, C, 1.0, jnp.float32)

  def do_deg_seg(seg):
    buf = ibufs[0]
    idx_start(seg, buf)
    idx_wait(buf)
    dstb = buf[1]

    def deg_body(j, _):
      stage_sidx(dstb, j)
      pltpu.sync_copy(rows1, acc_s.at[sidx], add=True)
      return 0

    lax.fori_loop(0, SB, deg_body, 0)

  @pl.when(c == 0)
  def _():
    for seg in (0, 1, 2):
      do_deg_seg(seg)

  @pl.when(c == 1)
  def _():
    for seg in (2, 3, 4):
      do_deg_seg(seg)

  plsc.subcore_barrier()

  @pl.when(c == 0)
  def _():
    writeback(deg_out.at[0])

  @pl.when(c == 1)
  def _():
    writeback(deg_out.at[1])


def _make_sc_segsum():
  return pl.kernel(
    _sc_body,
    out_type=(
        jax.ShapeDtypeStruct((NCHUNK, NP, C), jnp.float32),
        jax.ShapeDtypeStruct((NC, NP, C), jnp.float32),
    ),
    mesh=plsc.VectorSubcoreMesh(
        core_axis_name="c", subcore_axis_name="s",
        num_cores=NC, num_subcores=NS),
    scratch_types=[
        pltpu.VMEM((SEGE,), jnp.int32),
        pltpu.VMEM((SEGE,), jnp.int32),
        pltpu.VMEM((SEGE,), jnp.int32),
        pltpu.VMEM((SEGE,), jnp.int32),
        pltpu.VMEM((BB,), jnp.int32),
        pltpu.VMEM((BB, C), jnp.float32),
        pltpu.VMEM((BB, C), jnp.float32),
        pltpu.VMEM_SHARED((NP, C), jnp.float32),
        pltpu.SemaphoreType.DMA,
        pltpu.SemaphoreType.DMA,
        pltpu.SemaphoreType.DMA,
        pltpu.SemaphoreType.DMA,
    ],
  )


BN = 1000


def _table_body(h_ref, c_ref, ufw_ref, ufb_ref, t_ref):
  h = h_ref[...]
  f = jax.nn.sigmoid(
      lax.dot_general(h, ufw_ref[...], (((1,), (1,)), ((), ())),
                      precision=PREC) + ufb_ref[...])
  p = f * c_ref[...]
  t_ref[0] = h[:, :C]
  t_ref[1] = h[:, C:]
  t_ref[2] = p[:, :C]
  t_ref[3] = p[:, C:]


def _gates_body(x_ref, c_ref, acc_ref, deg_ref, wiou_ref, uiou_ref, biou_ref,
                h_out, c_out):
  acc = acc_ref[...]
  h_tild = jnp.concatenate([acc[0], acc[1]], axis=1)
  c_agg = jnp.concatenate([acc[2], acc[3]], axis=1)
  deg = deg_ref[0, :, :1] + deg_ref[1, :, :1]
  has_child = deg > 0.0
  uh = lax.dot_general(h_tild, uiou_ref[...], (((1,), (1,)), ((), ())),
                       precision=PREC)
  wx = lax.dot_general(x_ref[...], wiou_ref[...], (((1,), (1,)), ((), ())),
                       precision=PREC)
  iou = jnp.where(has_child, uh, wx) + biou_ref[...]
  i_g = jax.nn.sigmoid(iou[:, :H])
  o_g = jax.nn.sigmoid(iou[:, H:2 * H])
  u_g = jnp.tanh(iou[:, 2 * H:])
  c_data = jnp.where(has_child, c_agg, c_ref[...])
  c_new = i_g * u_g + c_data
  h_out[...] = o_g * jnp.tanh(c_new)
  c_out[...] = c_new


def kernel(x, h, c, edge_index, W_iou, U_iou, b_iou, U_f_w, U_f_b):
  src = edge_index[0]
  dst = edge_index[1]
  ufb = U_f_b.reshape(1, H)

  table = pl.pallas_call(
      _table_body,
      grid=(N // BN,),
      in_specs=[
          pl.BlockSpec((BN, H), lambda i: (i, 0)),
          pl.BlockSpec((BN, H), lambda i: (i, 0)),
          pl.BlockSpec((H, H), lambda i: (0, 0)),
          pl.BlockSpec((1, H), lambda i: (0, 0)),
      ],
      out_specs=pl.BlockSpec((NCHUNK, BN, C), lambda i: (0, i, 0)),
      out_shape=jax.ShapeDtypeStruct((NCHUNK, N, C), jnp.float32),
  )(h, c, U_f_w, ufb)

  acc, degp = _make_sc_segsum()(table, src, dst)

  h_new, c_new = pl.pallas_call(
      _gates_body,
      grid=(N // BN,),
      in_specs=[
          pl.BlockSpec((BN, H), lambda i: (i, 0)),
          pl.BlockSpec((BN, H), lambda i: (i, 0)),
          pl.BlockSpec((NCHUNK, BN, C), lambda i: (0, i, 0)),
          pl.BlockSpec((NC, BN, C), lambda i: (0, i, 0)),
          pl.BlockSpec((3 * H, H), lambda i: (0, 0)),
          pl.BlockSpec((3 * H, H), lambda i: (0, 0)),
          pl.BlockSpec((1, 3 * H), lambda i: (0, 0)),
      ],
      out_specs=[
          pl.BlockSpec((BN, H), lambda i: (i, 0)),
          pl.BlockSpec((BN, H), lambda i: (i, 0)),
      ],
      out_shape=[
          jax.ShapeDtypeStruct((N, H), jnp.float32),
          jax.ShapeDtypeStruct((N, H), jnp.float32),
      ],
  )(x, c, acc, degp, W_iou, U_iou, b_iou)

  return (h_new, c_new)


# speedup vs baseline: 6.6089x; 1.0234x over previous
import jax
import jax.numpy as jnp
from jax import lax
from jax.experimental import pallas as pl
from jax.experimental.pallas import tpu as pltpu
from jax.experimental.pallas import tpu_sc as plsc

N, E, H = 10000, 160000, 256
C = 128
NCHUNK = 4
NC, NS = 2, 16
EPT = E // NS
BB = 80
NBLK = EPT // BB
NSEG = 5
SB = NBLK // NSEG
SEGE = EPT // NSEG
DSEGS = (E // NC) // SEGE
NP = 10240
RPT = NP // NS
ZR = 80
WB = [(k * ZR, ZR) for k in range(RPT // ZR)]
DW = 16
PREC = lax.Precision.DEFAULT


def _fill2d(ref, rows, cols, value, dtype):
  vec = jnp.full((16,), value, dtype)

  def body(i, _):
    for j in range(cols // 16):
      ref[i, pl.ds(j * 16, 16)] = vec
    return 0

  lax.fori_loop(0, rows, body, 0)


def _sc_body(t_hbm, src_hbm, dst_hbm, acc_out, deg_out,
             src_a, dst_a, src_b, dst_b, sidx, rows0, rows1, zrows,
             acc_s, sem0, sem1, semi_a, semi_b):
  c = lax.axis_index("c")
  s = lax.axis_index("s")
  ibufs = ((src_a, dst_a, semi_a), (src_b, dst_b, semi_b))

  def idx_start(seg, buf):
    base = s * EPT + seg * SEGE
    srcb, dstb, semi = buf
    pltpu.async_copy(src_hbm.at[pl.ds(base, SEGE)], srcb, semi)
    pltpu.async_copy(dst_hbm.at[pl.ds(base, SEGE)], dstb, semi)

  def idx_wait(buf):
    srcb, dstb, semi = buf
    pltpu.make_async_copy(src_hbm.at[pl.ds(0, SEGE)], srcb, semi).wait()
    pltpu.make_async_copy(dst_hbm.at[pl.ds(0, SEGE)], dstb, semi).wait()

  def stage_sidx(dstb, j):
    for k in range(BB // 16):
      sidx[pl.ds(16 * k, 16)] = dstb[pl.ds(j * BB + 16 * k, 16)]

  _fill2d(zrows, ZR, C, 0.0, jnp.float32)

  def zero_acc():
    for off, n in WB:
      pltpu.sync_copy(zrows.at[pl.ds(0, n)],
                      acc_s.at[pl.ds(s * RPT + off, n)])
    plsc.subcore_barrier()

  def writeback(out_ref, and_zero):
    stags = (rows0, semi_a), (rows1, semi_b)
    for k, (off, n) in enumerate(WB):
      r = s * RPT + off
      stag, wsem = stags[k % 2]
      pltpu.sync_copy(acc_s.at[pl.ds(r, n)], stag.at[pl.ds(0, n)])
      if k >= 2:
        pltpu.make_async_copy(stag, out_ref.at[pl.ds(0, ZR)], wsem).wait()
      pltpu.async_copy(stag.at[pl.ds(0, n)], out_ref.at[pl.ds(r, n)], wsem)
      if and_zero:
        pltpu.sync_copy(zrows.at[pl.ds(0, n)], acc_s.at[pl.ds(r, n)])
    for stag, wsem in stags:
      pltpu.make_async_copy(stag, out_ref.at[pl.ds(0, ZR)], wsem).wait()
    plsc.subcore_barrier()

  def do_chunk(chunk):
    table = t_hbm.at[chunk]

    def gather_start(srcb, j, rows, sem):
      pltpu.async_copy(table.at[srcb.at[pl.ds(j * BB, BB)]], rows, sem)

    def gather_wait(rows, sem):
      pltpu.make_async_copy(table.at[src_a.at[pl.ds(0, BB)]], rows, sem).wait()

    idx_start(0, ibufs[0])

    for seg in range(NSEG):
      cur = ibufs[seg % 2]
      nxt = ibufs[(seg + 1) % 2]
      idx_wait(cur)
      if seg + 1 < NSEG:
        idx_start(seg + 1, nxt)
      srcb, dstb = cur[0], cur[1]

      gather_start(srcb, 0, rows0, sem0)
      gather_start(srcb, 1, rows1, sem1)

      def pair_body(g, _):
        for b, (rows, sem) in enumerate(((rows0, sem0), (rows1, sem1))):
          j = 2 * g + b
          gather_wait(rows, sem)
          gather_start(srcb, jnp.minimum(j + 2, SB - 1), rows, sem)
          stage_sidx(dstb, j)
          pltpu.sync_copy(rows, acc_s.at[sidx], add=True)
        return 0

      lax.fori_loop(0, (SB - 1) // 2, pair_body, 0)
      gather_wait(rows0, sem0)
      stage_sidx(dstb, SB - 1)
      pltpu.sync_copy(rows0, acc_s.at[sidx], add=True)
      gather_wait(rows1, sem1)

    plsc.subcore_barrier()
    writeback(acc_out.at[chunk], and_zero=True)

  zero_acc()

  @pl.when(c == 0)
  def _():
    do_chunk(0)
    do_chunk(1)

  @pl.when(c == 1)
  def _():
    do_chunk(2)
    do_chunk(3)

  _fill2d(rows1, BB, C, 1.0, jnp.float32)

  def do_deg_seg(seg):
    buf = ibufs[0]
    idx_start(seg, buf)
    idx_wait(buf)
    dstb = buf[1]

    def deg_body(j, _):
      stage_sidx(dstb, j)
      pltpu.sync_copy(rows1, acc_s.at[sidx], add=True)
      return 0

    lax.fori_loop(0, SB, deg_body, 0)

  @pl.when(c == 0)
  def _():
    for seg in (0, 1, 2):
      do_deg_seg(seg)

  @pl.when(c == 1)
  def _():
    for seg in (2, 3, 4):
      do_deg_seg(seg)

  plsc.subcore_barrier()

  @pl.when(c == 0)
  def _():
    writeback(deg_out.at[0], and_zero=False)

  @pl.when(c == 1)
  def _():
    writeback(deg_out.at[1], and_zero=False)


def _make_sc_segsum():
  return pl.kernel(
    _sc_body,
    out_type=(
        jax.ShapeDtypeStruct((NCHUNK, NP, C), jnp.float32),
        jax.ShapeDtypeStruct((NC, NP, C), jnp.float32),
    ),
    mesh=plsc.VectorSubcoreMesh(
        core_axis_name="c", subcore_axis_name="s",
        num_cores=NC, num_subcores=NS),
    scratch_types=[
        pltpu.VMEM((SEGE,), jnp.int32),
        pltpu.VMEM((SEGE,), jnp.int32),
        pltpu.VMEM((SEGE,), jnp.int32),
        pltpu.VMEM((SEGE,), jnp.int32),
        pltpu.VMEM((BB,), jnp.int32),
        pltpu.VMEM((BB, C), jnp.float32),
        pltpu.VMEM((BB, C), jnp.float32),
        pltpu.VMEM((ZR, C), jnp.float32),
        pltpu.VMEM_SHARED((NP, C), jnp.float32),
        pltpu.SemaphoreType.DMA,
        pltpu.SemaphoreType.DMA,
        pltpu.SemaphoreType.DMA,
        pltpu.SemaphoreType.DMA,
    ],
  )


BN = 1000
BNG = 1024


def _table_body(h_ref, c_ref, ufw_ref, ufb_ref, t_ref):
  h = h_ref[...]
  f = jax.nn.sigmoid(
      lax.dot_general(h, ufw_ref[...], (((1,), (1,)), ((), ())),
                      precision=PREC) + ufb_ref[...])
  p = f * c_ref[...]
  t_ref[0] = h[:, :C]
  t_ref[1] = h[:, C:]
  t_ref[2] = p[:, :C]
  t_ref[3] = p[:, C:]


def _gates_body(x_ref, c_ref, acc_ref, deg_ref, wiou_ref, uiou_ref, biou_ref,
                h_out, c_out):
  acc = acc_ref[...]
  h_tild = jnp.concatenate([acc[0], acc[1]], axis=1)
  c_agg = jnp.concatenate([acc[2], acc[3]], axis=1)
  deg = deg_ref[0, :, :1] + deg_ref[1, :, :1]
  has_child = deg > 0.0
  uh = lax.dot_general(h_tild, uiou_ref[...], (((1,), (1,)), ((), ())),
                       precision=PREC)
  wx = lax.dot_general(x_ref[...], wiou_ref[...], (((1,), (1,)), ((), ())),
                       precision=PREC)
  iou = jnp.where(has_child, uh, wx) + biou_ref[...]
  i_g = jax.nn.sigmoid(iou[:, :H])
  o_g = jax.nn.sigmoid(iou[:, H:2 * H])
  u_g = jnp.tanh(iou[:, 2 * H:])
  c_data = jnp.where(has_child, c_agg, c_ref[...])
  c_new = i_g * u_g + c_data
  h_out[...] = o_g * jnp.tanh(c_new)
  c_out[...] = c_new


def kernel(x, h, c, edge_index, W_iou, U_iou, b_iou, U_f_w, U_f_b):
  src = edge_index[0]
  dst = edge_index[1]
  ufb = U_f_b.reshape(1, H)

  table = pl.pallas_call(
      _table_body,
      grid=(N // BN,),
      in_specs=[
          pl.BlockSpec((BN, H), lambda i: (i, 0)),
          pl.BlockSpec((BN, H), lambda i: (i, 0)),
          pl.BlockSpec((H, H), lambda i: (0, 0)),
          pl.BlockSpec((1, H), lambda i: (0, 0)),
      ],
      out_specs=pl.BlockSpec((NCHUNK, BN, C), lambda i: (0, i, 0)),
      out_shape=jax.ShapeDtypeStruct((NCHUNK, N, C), jnp.float32),
  )(h, c, U_f_w, ufb)

  acc, degp = _make_sc_segsum()(table, src, dst)

  h_new, c_new = pl.pallas_call(
      _gates_body,
      grid=(NP // BNG,),
      in_specs=[
          pl.BlockSpec((BNG, H), lambda i: (i, 0)),
          pl.BlockSpec((BNG, H), lambda i: (i, 0)),
          pl.BlockSpec((NCHUNK, BNG, C), lambda i: (0, i, 0)),
          pl.BlockSpec((NC, BNG, C), lambda i: (0, i, 0)),
          pl.BlockSpec((3 * H, H), lambda i: (0, 0)),
          pl.BlockSpec((3 * H, H), lambda i: (0, 0)),
          pl.BlockSpec((1, 3 * H), lambda i: (0, 0)),
      ],
      out_specs=[
          pl.BlockSpec((BNG, H), lambda i: (i, 0)),
          pl.BlockSpec((BNG, H), lambda i: (i, 0)),
      ],
      out_shape=[
          jax.ShapeDtypeStruct((N, H), jnp.float32),
          jax.ShapeDtypeStruct((N, H), jnp.float32),
      ],
  )(x, c, acc, degp, W_iou, U_iou, b_iou)

  return (h_new, c_new)
